# baseline (speedup 1.0000x reference)
import functools

import jax
import jax.numpy as jnp
from jax import lax
from jax.experimental import pallas as pl
from jax.experimental.pallas import tpu as pltpu
from jax.experimental.pallas import tpu_sc as plsc

VOCAB_DIM = 1000000
EMBD_DIM = 64
N = 819200

NUM_CORES = 2
NUM_SUBCORES = 16
NW = NUM_CORES * NUM_SUBCORES
GROUP = 128
GROUPS_PER_W = N // (NW * GROUP)
K = 8

_mesh = plsc.VectorSubcoreMesh(core_axis_name="c", subcore_axis_name="s")


@functools.partial(
    pl.kernel,
    mesh=_mesh,
    compiler_params=pltpu.CompilerParams(use_tc_tiling_on_sc=False),
    out_type=jax.ShapeDtypeStruct((NW, GROUPS_PER_W, GROUP, EMBD_DIM), jnp.float32),
    scratch_types=[
        pltpu.VMEM((GROUPS_PER_W, GROUP), jnp.int32),
        pltpu.VMEM((K, GROUP, EMBD_DIM), jnp.float32),
        pltpu.SemaphoreType.DMA,
        pltpu.SemaphoreType.DMA,
    ],
)
def _embed_gather(table_hbm, idx_hbm, out_hbm, idx_v, rows_v, gsem, osem):
    wid = lax.axis_index("s") * NUM_CORES + lax.axis_index("c")
    pltpu.sync_copy(idx_hbm.at[wid], idx_v)

    def body(t, carry):
        gathers = []
        for b in range(K):
            g = t * K + b
            gathers.append(
                pltpu.async_copy(table_hbm.at[idx_v.at[g]], rows_v.at[b], gsem)
            )
        outs = []
        for b in range(K):
            g = t * K + b
            gathers[b].wait()
            outs.append(pltpu.async_copy(rows_v.at[b], out_hbm.at[wid, g], osem))
        for o in outs:
            o.wait()
        return carry

    lax.fori_loop(0, GROUPS_PER_W // K, body, 0)


def kernel(x, vocab_table):
    idx = x.reshape(NW, GROUPS_PER_W, GROUP)
    out = _embed_gather(vocab_table, idx)
    return out.reshape(N, EMBD_DIM)


# speedup vs baseline: 1.0017x; 1.0017x over previous
import functools

import jax
import jax.numpy as jnp
from jax import lax
from jax.experimental import pallas as pl
from jax.experimental.pallas import tpu as pltpu
from jax.experimental.pallas import tpu_sc as plsc

VOCAB_DIM = 1000000
EMBD_DIM = 64
N = 819200

NUM_CORES = 2
NUM_SUBCORES = 16
NW = NUM_CORES * NUM_SUBCORES
GROUP = 128
GROUPS_PER_W = N // (NW * GROUP)
CH = 4
NCHUNK = GROUPS_PER_W // CH
NBUF = 3

_mesh = plsc.VectorSubcoreMesh(core_axis_name="c", subcore_axis_name="s")


@functools.partial(
    pl.kernel,
    mesh=_mesh,
    compiler_params=pltpu.CompilerParams(use_tc_tiling_on_sc=False),
    out_type=jax.ShapeDtypeStruct((NW, GROUPS_PER_W, GROUP, EMBD_DIM), jnp.float32),
    scratch_types=[
        pltpu.VMEM((GROUPS_PER_W, GROUP), jnp.int32),
        pltpu.VMEM((NBUF, CH, GROUP, EMBD_DIM), jnp.float32),
        pltpu.SemaphoreType.DMA,
        pltpu.SemaphoreType.DMA,
    ],
)
def _embed_gather(table_hbm, idx_hbm, out_hbm, idx_v, rows_v, gsem, osem):
    wid = lax.axis_index("s") * NUM_CORES + lax.axis_index("c")
    pltpu.sync_copy(idx_hbm.at[wid], idx_v)

    def fire_chunk(c, slot):
        for b in range(CH):
            pltpu.async_copy(
                table_hbm.at[idx_v.at[c * CH + b]], rows_v.at[slot, b], gsem
            )

    def wait_chunk(c, slot):
        for b in range(CH):
            pltpu.make_async_copy(
                table_hbm.at[idx_v.at[c * CH + b]], rows_v.at[slot, b], gsem
            ).wait()

    def fire_out(c, slot):
        pltpu.async_copy(rows_v.at[slot], out_hbm.at[wid, pl.ds(c * CH, CH)], osem)

    def wait_out(c, slot):
        pltpu.make_async_copy(
            rows_v.at[slot], out_hbm.at[wid, pl.ds(c * CH, CH)], osem
        ).wait()

    fire_chunk(0, 0)

    def body(t, carry):
        s_cur = lax.rem(t, NBUF)
        s_nxt = lax.rem(t + 1, NBUF)

        @pl.when(t >= NBUF - 1)
        def _():
            wait_out(t - (NBUF - 1), s_nxt)

        @pl.when(t <= NCHUNK - 2)
        def _():
            fire_chunk(t + 1, s_nxt)

        wait_chunk(t, s_cur)
        fire_out(t, s_cur)
        return carry

    lax.fori_loop(0, NCHUNK, body, 0)

    for c in range(NCHUNK - (NBUF - 1), NCHUNK):
        wait_out(c, c % NBUF)


def kernel(x, vocab_table):
    idx = x.reshape(NW, GROUPS_PER_W, GROUP)
    out = _embed_gather(vocab_table, idx)
    return out.reshape(N, EMBD_DIM)


# speedup vs baseline: 1.0019x; 1.0002x over previous
import functools

import jax
import jax.numpy as jnp
from jax import lax
from jax.experimental import pallas as pl
from jax.experimental.pallas import tpu as pltpu
from jax.experimental.pallas import tpu_sc as plsc

VOCAB_DIM = 1000000
EMBD_DIM = 64
N = 819200

NUM_CORES = 2
NUM_SUBCORES = 16
NW = NUM_CORES * NUM_SUBCORES
GROUP = 128
ROWS_PER_W = N // NW
CH = 4
CHROWS = CH * GROUP
NCHUNK = ROWS_PER_W // CHROWS
NBUF = 3

_mesh = plsc.VectorSubcoreMesh(core_axis_name="c", subcore_axis_name="s")


@functools.partial(
    pl.kernel,
    mesh=_mesh,
    compiler_params=pltpu.CompilerParams(use_tc_tiling_on_sc=False),
    out_type=jax.ShapeDtypeStruct((N, EMBD_DIM), jnp.float32),
    scratch_types=[
        pltpu.VMEM((ROWS_PER_W,), jnp.int32),
        pltpu.VMEM((NBUF, CHROWS, EMBD_DIM), jnp.float32),
        pltpu.SemaphoreType.DMA,
        pltpu.SemaphoreType.DMA,
    ],
)
def _embed_gather(table_hbm, idx_hbm, out_hbm, idx_v, rows_v, gsem, osem):
    wid = lax.axis_index("s") * NUM_CORES + lax.axis_index("c")
    base = wid * ROWS_PER_W
    pltpu.sync_copy(idx_hbm.at[pl.ds(base, ROWS_PER_W)], idx_v)

    def fire_chunk(c, slot):
        for b in range(CH):
            pltpu.async_copy(
                table_hbm.at[idx_v.at[pl.ds(c * CHROWS + b * GROUP, GROUP)]],
                rows_v.at[slot, pl.ds(b * GROUP, GROUP)],
                gsem,
            )

    def wait_chunk(c, slot):
        for b in range(CH):
            pltpu.make_async_copy(
                table_hbm.at[idx_v.at[pl.ds(c * CHROWS + b * GROUP, GROUP)]],
                rows_v.at[slot, pl.ds(b * GROUP, GROUP)],
                gsem,
            ).wait()

    def fire_out(c, slot):
        pltpu.async_copy(
            rows_v.at[slot], out_hbm.at[pl.ds(base + c * CHROWS, CHROWS)], osem
        )

    def wait_out(c, slot):
        pltpu.make_async_copy(
            rows_v.at[slot], out_hbm.at[pl.ds(base + c * CHROWS, CHROWS)], osem
        ).wait()

    fire_chunk(0, 0)

    def body(t, carry):
        s_cur = lax.rem(t, NBUF)
        s_nxt = lax.rem(t + 1, NBUF)

        @pl.when(t >= NBUF - 1)
        def _():
            wait_out(t - (NBUF - 1), s_nxt)

        @pl.when(t <= NCHUNK - 2)
        def _():
            fire_chunk(t + 1, s_nxt)

        wait_chunk(t, s_cur)
        fire_out(t, s_cur)
        return carry

    lax.fori_loop(0, NCHUNK, body, 0)

    for c in range(NCHUNK - (NBUF - 1), NCHUNK):
        wait_out(c, c % NBUF)


def kernel(x, vocab_table):
    return _embed_gather(vocab_table, x.reshape(N))


# speedup vs baseline: 2.4807x; 2.4760x over previous
import functools

import jax
import jax.numpy as jnp
from jax import lax
from jax.experimental import pallas as pl
from jax.experimental.pallas import tpu as pltpu
from jax.experimental.pallas import tpu_sc as plsc

VOCAB_DIM = 1000000
EMBD_DIM = 64
N = 819200

NUM_CORES = 2
NUM_SUBCORES = 16
NW = NUM_CORES * NUM_SUBCORES
GROUP = 128
ROWS_PER_W = N // NW

_mesh = plsc.VectorSubcoreMesh(core_axis_name="c", subcore_axis_name="s")


FCH = 256
FNS = ROWS_PER_W // FCH
OPAD = 131


@functools.partial(
    pl.kernel,
    mesh=_mesh,
    compiler_params=pltpu.CompilerParams(
        use_tc_tiling_on_sc=False, needs_layout_passes=False
    ),
    out_type=jax.ShapeDtypeStruct((8, N // 128, 8, 128), jnp.float32),
    scratch_types=[
        pltpu.VMEM((ROWS_PER_W,), jnp.int32),
        pltpu.VMEM((3, FCH, EMBD_DIM), jnp.float32),
        pltpu.VMEM((2, 2, EMBD_DIM, OPAD), jnp.float32),
        pltpu.SemaphoreType.DMA,
        pltpu.SemaphoreType.DMA,
    ],
)
def _gather_transpose(table_hbm, idx_hbm, out4_hbm, idx_v, g_v, o_v, gsem, osem):
    wid = lax.axis_index("s") * NUM_CORES + lax.axis_index("c")
    base = wid * ROWS_PER_W
    tc0 = wid * (ROWS_PER_W // 128)
    pltpu.sync_copy(idx_hbm.at[pl.ds(base, ROWS_PER_W)], idx_v)

    lanes = lax.iota(jnp.int32, 16)

    def fire_gather(s, slot):
        for b in range(FCH // GROUP):
            pltpu.async_copy(
                table_hbm.at[idx_v.at[pl.ds(s * FCH + b * GROUP, GROUP)]],
                g_v.at[slot, pl.ds(b * GROUP, GROUP)],
                gsem,
            )

    def wait_gather(s, slot):
        for b in range(FCH // GROUP):
            pltpu.make_async_copy(
                table_hbm.at[idx_v.at[pl.ds(s * FCH + b * GROUP, GROUP)]],
                g_v.at[slot, pl.ds(b * GROUP, GROUP)],
                gsem,
            ).wait()

    def fire_out(s, slot):
        for r in range(8):
            pltpu.async_copy(
                o_v.at[slot, :, pl.ds(8 * r, 8), pl.ds(0, 128)],
                out4_hbm.at[r, pl.ds(tc0 + 2 * s, 2)],
                osem,
            )

    def wait_out(s, slot):
        for r in range(8):
            pltpu.make_async_copy(
                o_v.at[slot, :, pl.ds(8 * r, 8), pl.ds(0, 128)],
                out4_hbm.at[r, pl.ds(tc0 + 2 * s, 2)],
                osem,
            ).wait()

    fire_gather(0, 0)
    fire_gather(1, 1)
    fire_gather(2, 2)

    def body(s, carry):
        s3 = lax.rem(s, 3)
        s2 = lax.rem(s, 2)
        wait_gather(s, s3)

        @pl.when(s >= 2)
        def _():
            wait_out(s - 2, s2)

        for kk in range(2):

            @plsc.parallel_loop(0, GROUP, unroll=2)
            def pbody(p2):
                pvec = jnp.full((16,), p2, dtype=jnp.int32)
                for q in range(EMBD_DIM // 16):
                    v = g_v[s3, 128 * kk + p2, pl.ds(16 * q, 16)]
                    plsc.store_scatter(
                        o_v.at[s2, kk], [16 * q + lanes, pvec], v
                    )

        fire_out(s, s2)

        @pl.when(s <= FNS - 4)
        def _():
            fire_gather(s + 3, s3)

        return carry

    lax.fori_loop(0, FNS, body, 0)
    wait_out(FNS - 2, 0)
    wait_out(FNS - 1, 1)


VTC = 7813
VSUP = (VTC - 1) // 2
VS_BASE = VSUP // NW
VS_EXTRA = VSUP - VS_BASE * NW


@functools.partial(
    pl.kernel,
    mesh=_mesh,
    compiler_params=pltpu.CompilerParams(
        use_tc_tiling_on_sc=False, needs_layout_passes=False
    ),
    out_type=jax.ShapeDtypeStruct((500032, 128), jnp.float32),
    scratch_types=[
        pltpu.VMEM((3, 2, EMBD_DIM, OPAD), jnp.float32),
        pltpu.VMEM((2, 128, 128), jnp.float32),
        pltpu.SemaphoreType.DMA,
        pltpu.SemaphoreType.DMA,
    ],
)
def _untile_table(tab4_hbm, rm_hbm, in_v, out_v, isem, osem):
    wid = lax.axis_index("s") * NUM_CORES + lax.axis_index("c")
    base = VS_BASE * wid + jnp.minimum(wid, VS_EXTRA)
    count = VS_BASE + jnp.where(wid < VS_EXTRA, 1, 0)

    lanes = lax.iota(jnp.int32, 16)

    def fire_in(s, slot):
        for r in range(8):
            pltpu.async_copy(
                tab4_hbm.at[r, pl.ds((base + s) * 2, 2)],
                in_v.at[slot, :, pl.ds(8 * r, 8), pl.ds(0, 128)],
                isem,
            )

    def wait_in(s, slot):
        for r in range(8):
            pltpu.make_async_copy(
                tab4_hbm.at[r, pl.ds((base + s) * 2, 2)],
                in_v.at[slot, :, pl.ds(8 * r, 8), pl.ds(0, 128)],
                isem,
            ).wait()

    def fire_out(s, slot):
        pltpu.async_copy(
            out_v.at[slot], rm_hbm.at[pl.ds((base + s) * 128, 128)], osem
        )

    def wait_out(s, slot):
        pltpu.make_async_copy(
            out_v.at[slot], rm_hbm.at[pl.ds((base + s) * 128, 128)], osem
        ).wait()

    fire_in(0, 0)
    fire_in(1, 1)
    fire_in(2, 2)

    def body(s, carry):
        s3 = lax.rem(s, 3)
        slot = lax.rem(s, 2)
        wait_in(s, s3)

        @pl.when(s >= 2)
        def _():
            wait_out(s - 2, slot)

        for kk in range(2):

            @plsc.parallel_loop(0, EMBD_DIM, unroll=2)
            def abody(a):
                for m in range(8):
                    rows = 16 * (m % 4) + lanes
                    cvec = jnp.full((16,), 2 * a + m // 4, dtype=jnp.int32)
                    v = plsc.load_gather(in_v.at[s3, kk], [rows, cvec])
                    out_v[slot, 64 * kk + a, pl.ds(16 * m, 16)] = v

        fire_out(s, slot)

        @pl.when(s <= count - 4)
        def _():
            fire_in(s + 3, s3)

        return carry

    lax.fori_loop(0, count, body, 0)
    wait_out(count - 2, lax.rem(count - 2, 2))
    wait_out(count - 1, lax.rem(count - 1, 2))

    @pl.when(wid == NW - 1)
    def _():
        for r in range(8):
            pltpu.sync_copy(
                tab4_hbm.at[r, pl.ds(VTC - 1, 1)],
                in_v.at[0, pl.ds(0, 1), pl.ds(8 * r, 8), pl.ds(0, 128)],
            )

        @plsc.parallel_loop(0, EMBD_DIM, unroll=2)
        def tbody(a):
            for m in range(8):
                rows = 16 * (m % 4) + lanes
                cvec = jnp.full((16,), 2 * a + m // 4, dtype=jnp.int32)
                v = plsc.load_gather(in_v.at[0, 0], [rows, cvec])
                out_v[0, a, pl.ds(16 * m, 16)] = v

        pltpu.sync_copy(
            out_v.at[0, pl.ds(0, EMBD_DIM)],
            rm_hbm.at[pl.ds((VTC - 1) * 64, EMBD_DIM)],
        )


def kernel(x, vocab_table):
    tab4 = (
        jnp.pad(vocab_table, ((0, 64), (0, 0)))
        .T.reshape(8, 8, VTC, 128)
        .transpose(0, 2, 1, 3)
    )
    rm = _untile_table(tab4)
    out4 = _gather_transpose(rm.reshape(1000064, EMBD_DIM), x.reshape(N))
    return jnp.transpose(out4, (1, 3, 0, 2)).reshape(N, EMBD_DIM)
